# baseline (speedup 1.0000x reference)
import functools

import jax
import jax.numpy as jnp
from jax import lax
from jax.experimental import pallas as pl
from jax.experimental.pallas import tpu as pltpu

_T = 16
_S_CHUNK = 64
_N_PAR = 2
_UNROLL = 8


def _lif_body(x_ref, y_ref, mem_ref, res_ref, *, s_chunk, unroll):
    sc = pl.program_id(1)

    @pl.when(sc == 0)
    def _init():
        mem_ref[...] = jnp.zeros_like(mem_ref)
        res_ref[...] = jnp.zeros_like(res_ref)

    one = jnp.float32(1.0)
    zero = jnp.float32(0.0)

    def step(t, carry):
        m, r = carry
        m = m + x_ref[t]
        c = m >= one
        r = 0.5 * r + jnp.where(c, one, zero)
        y_ref[t] = r
        m = jnp.where(c, zero, m)
        return m, r

    m, r = lax.fori_loop(0, s_chunk, step, (mem_ref[...], res_ref[...]),
                         unroll=unroll)
    mem_ref[...] = m
    res_ref[...] = r


def kernel(x):
    steps, TB, D = x.shape
    B = TB // _T
    S = steps * _T

    xk = x.reshape(S, B, D)

    tb = B // _N_PAR if B % _N_PAR == 0 else B
    n_b = B // tb
    s_chunk = _S_CHUNK if S % _S_CHUNK == 0 else S
    n_s = S // s_chunk

    body = functools.partial(_lif_body, s_chunk=s_chunk, unroll=_UNROLL)

    y, mem, res = pl.pallas_call(
        body,
        out_shape=(
            jax.ShapeDtypeStruct((S, B, D), jnp.float32),
            jax.ShapeDtypeStruct((B, D), jnp.float32),
            jax.ShapeDtypeStruct((B, D), jnp.float32),
        ),
        grid=(n_b, n_s),
        in_specs=[pl.BlockSpec((s_chunk, tb, D), lambda j, s: (s, j, 0))],
        out_specs=(
            pl.BlockSpec((s_chunk, tb, D), lambda j, s: (s, j, 0)),
            pl.BlockSpec((tb, D), lambda j, s: (j, 0)),
            pl.BlockSpec((tb, D), lambda j, s: (j, 0)),
        ),
        compiler_params=pltpu.CompilerParams(
            dimension_semantics=("parallel", "arbitrary"),
            vmem_limit_bytes=64 * 1024 * 1024,
        ),
    )(xk)

    return y.reshape(steps, TB, D), mem, res


# speedup vs baseline: 1.0633x; 1.0633x over previous
import functools

import jax
import jax.numpy as jnp
from jax import lax
from jax.experimental import pallas as pl
from jax.experimental.pallas import tpu as pltpu

_T = 16
_S_CHUNK = 64
_N_PAR = 2
_K_IN = 3
_K_OUT = 3
_UNROLL = 8


def _lif_body(x_hbm, y_hbm, mem_ref, res_ref, in_buf, out_buf, in_sem,
              out_sem, *, sc, n_c, tb, unroll):
    j = pl.program_id(0)
    b0 = j * tb

    def in_copy(c, slot):
        return pltpu.make_async_copy(
            x_hbm.at[pl.ds(c * sc, sc), pl.ds(b0, tb), :],
            in_buf.at[slot], in_sem.at[slot])

    def out_copy(c, slot):
        return pltpu.make_async_copy(
            out_buf.at[slot],
            y_hbm.at[pl.ds(c * sc, sc), pl.ds(b0, tb), :],
            out_sem.at[slot])

    for c in range(min(_K_IN, n_c)):
        in_copy(c, c % _K_IN).start()

    one = jnp.float32(1.0)
    zero = jnp.float32(0.0)

    def chunk(c, carry):
        m, r = carry
        islot = lax.rem(c, _K_IN)
        oslot = lax.rem(c, _K_OUT)
        in_copy(c, islot).wait()

        @pl.when(c >= _K_OUT)
        def _():
            out_copy(c - _K_OUT, oslot).wait()

        def step(t, mr):
            m, r = mr
            m = m + in_buf[islot, t]
            cnd = m >= one
            r = 0.5 * r + jnp.where(cnd, one, zero)
            out_buf[oslot, t] = r
            m = jnp.where(cnd, zero, m)
            return m, r

        m, r = lax.fori_loop(0, sc, step, (m, r), unroll=unroll)

        out_copy(c, oslot).start()
        @pl.when(c + _K_IN < n_c)
        def _():
            in_copy(c + _K_IN, lax.rem(c + _K_IN, _K_IN)).start()
        return m, r

    zeros = jnp.zeros((tb, x_hbm.shape[2]), jnp.float32)
    m, r = lax.fori_loop(0, n_c, chunk, (zeros, zeros))
    mem_ref[...] = m
    res_ref[...] = r

    tail = min(_K_OUT, n_c)
    for i in range(tail):
        c = n_c - tail + i
        out_copy(c, c % _K_OUT).wait()


def kernel(x):
    steps, TB, D = x.shape
    B = TB // _T
    S = steps * _T

    xk = x.reshape(S, B, D)

    tb = B // _N_PAR if B % _N_PAR == 0 else B
    n_b = B // tb
    sc = _S_CHUNK if S % _S_CHUNK == 0 else S
    n_c = S // sc

    body = functools.partial(_lif_body, sc=sc, n_c=n_c, tb=tb,
                             unroll=_UNROLL)

    y, mem, res = pl.pallas_call(
        body,
        out_shape=(
            jax.ShapeDtypeStruct((S, B, D), jnp.float32),
            jax.ShapeDtypeStruct((B, D), jnp.float32),
            jax.ShapeDtypeStruct((B, D), jnp.float32),
        ),
        grid=(n_b,),
        in_specs=[pl.BlockSpec(memory_space=pl.ANY)],
        out_specs=(
            pl.BlockSpec(memory_space=pl.ANY),
            pl.BlockSpec((tb, D), lambda j: (j, 0)),
            pl.BlockSpec((tb, D), lambda j: (j, 0)),
        ),
        scratch_shapes=[
            pltpu.VMEM((_K_IN, sc, tb, D), jnp.float32),
            pltpu.VMEM((_K_OUT, sc, tb, D), jnp.float32),
            pltpu.SemaphoreType.DMA((_K_IN,)),
            pltpu.SemaphoreType.DMA((_K_OUT,)),
        ],
        compiler_params=pltpu.CompilerParams(
            dimension_semantics=("parallel",),
            vmem_limit_bytes=64 * 1024 * 1024,
        ),
    )(xk)

    return y.reshape(steps, TB, D), mem, res


# speedup vs baseline: 1.0882x; 1.0235x over previous
import functools

import jax
import jax.numpy as jnp
from jax import lax
from jax.experimental import pallas as pl
from jax.experimental.pallas import tpu as pltpu

_T = 16
_SC_MAX = 64
_N_PAR = 2
_K_IN = 3
_K_OUT = 3
_UNROLL = 8


def _chunk_plan(S):
    if S <= 48 or S % 16 != 0:
        return [S]
    sizes = [16, 32]
    rem = S - 48
    while rem > _SC_MAX + 16:
        sizes.append(_SC_MAX)
        rem -= _SC_MAX
    if rem > _SC_MAX:
        sizes.append(rem - 16)
        rem = 16
    sizes.append(rem)
    return sizes


def _lif_body(x_hbm, y_hbm, mem_ref, res_ref, in_buf, out_buf, in_sem,
              out_sem, *, sizes, offs, tb, D, unroll):
    j = pl.program_id(0)
    b0 = j * tb
    n_c = len(sizes)

    def in_copy(c):
        return pltpu.make_async_copy(
            x_hbm.at[pl.ds(offs[c], sizes[c]), pl.ds(b0, tb), :],
            in_buf.at[c % _K_IN, pl.ds(0, sizes[c])],
            in_sem.at[c % _K_IN])

    def out_copy(c):
        return pltpu.make_async_copy(
            out_buf.at[c % _K_OUT, pl.ds(0, sizes[c])],
            y_hbm.at[pl.ds(offs[c], sizes[c]), pl.ds(b0, tb), :],
            out_sem.at[c % _K_OUT])

    for c in range(min(_K_IN, n_c)):
        in_copy(c).start()

    one = jnp.float32(1.0)
    zero = jnp.float32(0.0)

    m = jnp.zeros((tb, D), jnp.float32)
    r = jnp.zeros((tb, D), jnp.float32)

    for c in range(n_c):
        islot = c % _K_IN
        oslot = c % _K_OUT
        in_copy(c).wait()
        if c >= _K_OUT:
            out_copy(c - _K_OUT).wait()

        def step(t, mr, islot=islot, oslot=oslot):
            m, r = mr
            m = m + in_buf[islot, t]
            cnd = m >= one
            r = 0.5 * r + jnp.where(cnd, one, zero)
            out_buf[oslot, t] = r
            m = jnp.where(cnd, zero, m)
            return m, r

        m, r = lax.fori_loop(0, sizes[c], step, (m, r), unroll=unroll)

        out_copy(c).start()
        if c + _K_IN < n_c:
            in_copy(c + _K_IN).start()

    mem_ref[...] = m
    res_ref[...] = r

    for c in range(max(0, n_c - _K_OUT), n_c):
        out_copy(c).wait()


def kernel(x):
    steps, TB, D = x.shape
    B = TB // _T
    S = steps * _T

    xk = x.reshape(S, B, D)

    tb = B // _N_PAR if B % _N_PAR == 0 else B
    n_b = B // tb
    sizes = _chunk_plan(S)
    offs = [0]
    for s in sizes:
        offs.append(offs[-1] + s)
    sc_max = max(sizes)

    body = functools.partial(_lif_body, sizes=sizes, offs=offs, tb=tb, D=D,
                             unroll=_UNROLL)

    y, mem, res = pl.pallas_call(
        body,
        out_shape=(
            jax.ShapeDtypeStruct((S, B, D), jnp.float32),
            jax.ShapeDtypeStruct((B, D), jnp.float32),
            jax.ShapeDtypeStruct((B, D), jnp.float32),
        ),
        grid=(n_b,),
        in_specs=[pl.BlockSpec(memory_space=pl.ANY)],
        out_specs=(
            pl.BlockSpec(memory_space=pl.ANY),
            pl.BlockSpec((tb, D), lambda j: (j, 0)),
            pl.BlockSpec((tb, D), lambda j: (j, 0)),
        ),
        scratch_shapes=[
            pltpu.VMEM((_K_IN, sc_max, tb, D), jnp.float32),
            pltpu.VMEM((_K_OUT, sc_max, tb, D), jnp.float32),
            pltpu.SemaphoreType.DMA((_K_IN,)),
            pltpu.SemaphoreType.DMA((_K_OUT,)),
        ],
        compiler_params=pltpu.CompilerParams(
            dimension_semantics=("parallel",),
            vmem_limit_bytes=64 * 1024 * 1024,
        ),
    )(xk)

    return y.reshape(steps, TB, D), mem, res
